# Initial kernel scaffold; baseline (speedup 1.0000x reference)
import jax
import jax.numpy as jnp
from jax.experimental import pallas as pl


def kernel(input_pos, k_val, v_val, k_cache, v_cache):
    raise NotImplementedError("write your pallas kernel here")


# speedup vs baseline: 1.0101x; 1.0101x over previous
import functools

import jax
import jax.numpy as jnp
from jax.experimental import pallas as pl
from jax.experimental.pallas import tpu as pltpu

B, H, S, D = 8, 16, 2048, 128
L = 16
BH = B * H


def _body(pos_ref, kc, vc, kv, vv, ko, vo):
    ko[...] = kc[...]
    vo[...] = vc[...]
    for i in range(L):
        r = pos_ref[i]
        ko[0, pl.ds(r, 1), :] = kv[0, pl.ds(i, 1), :]
        vo[0, pl.ds(r, 1), :] = vv[0, pl.ds(i, 1), :]


@jax.jit
def _run(input_pos, k_val, v_val, k_cache, v_cache):
    kc = k_cache.reshape(BH, S, D)
    vc = v_cache.reshape(BH, S, D)
    kv = k_val.reshape(BH, L, D)
    vv = v_val.reshape(BH, L, D)

    grid_spec = pltpu.PrefetchScalarGridSpec(
        num_scalar_prefetch=1,
        grid=(BH,),
        in_specs=[
            pl.BlockSpec((1, S, D), lambda i, pos: (i, 0, 0)),
            pl.BlockSpec((1, S, D), lambda i, pos: (i, 0, 0)),
            pl.BlockSpec((1, L, D), lambda i, pos: (i, 0, 0)),
            pl.BlockSpec((1, L, D), lambda i, pos: (i, 0, 0)),
        ],
        out_specs=[
            pl.BlockSpec((1, S, D), lambda i, pos: (i, 0, 0)),
            pl.BlockSpec((1, S, D), lambda i, pos: (i, 0, 0)),
        ],
    )
    ko, vo = pl.pallas_call(
        _body,
        grid_spec=grid_spec,
        out_shape=[
            jax.ShapeDtypeStruct((BH, S, D), jnp.float32),
            jax.ShapeDtypeStruct((BH, S, D), jnp.float32),
        ],
    )(input_pos, kc, vc, kv, vv)
    return ko.reshape(B, H, S, D), vo.reshape(B, H, S, D)


def kernel(input_pos, k_val, v_val, k_cache, v_cache):
    return _run(input_pos, k_val, v_val, k_cache, v_cache)
